# baseline (speedup 1.0000x reference)
import functools

import jax
import jax.numpy as jnp
from jax import lax
from jax.experimental import pallas as pl
from jax.experimental.pallas import tpu as pltpu
from jax.experimental.pallas import tpu_sc as plsc

N_LABELS = 1000
PAD_LABELS = 1024
EMBED = 64
BATCH = 4096
T_STEPS = 200
T_OUT = T_STEPS - 1
T_PAD = 208
LANES = 16
NUM_CORES = 2
NUM_SUBCORES = 16
NW = NUM_CORES * NUM_SUBCORES
ROWS_W = BATCH // NW
SEQ_W = ROWS_W * T_STEPS
IDX_W = ROWS_W * T_PAD
STREAM_LEN = 128
N_STREAM = IDX_W // STREAM_LEN
LAG = 8
ROW_BLOCK = 128

NEG = -1e30


def _tc_table_body(s_ref, t_ref, o_ref):
    s = s_ref[...]
    t = t_ref[...]
    logits = lax.dot_general(
        s, t, (((1,), (1,)), ((), ())),
        preferred_element_type=jnp.float32,
        precision=lax.Precision.HIGHEST,
    )
    col = lax.broadcasted_iota(jnp.int32, logits.shape, 1)
    logits = jnp.where(col < N_LABELS, logits, NEG)
    m = jnp.max(logits, axis=1, keepdims=True)
    lse = jnp.log(jnp.sum(jnp.exp(logits - m), axis=1, keepdims=True)) + m
    o_ref[...] = logits - lse


def _tc_table(s_pad, t_pad):
    return pl.pallas_call(
        _tc_table_body,
        grid=(PAD_LABELS // ROW_BLOCK,),
        in_specs=[
            pl.BlockSpec((ROW_BLOCK, EMBED), lambda i: (i, 0)),
            pl.BlockSpec((PAD_LABELS, EMBED), lambda i: (0, 0)),
        ],
        out_specs=pl.BlockSpec((ROW_BLOCK, PAD_LABELS), lambda i: (i, 0)),
        out_shape=jax.ShapeDtypeStruct((PAD_LABELS, PAD_LABELS), jnp.float32),
    )(s_pad, t_pad)


@functools.partial(
    pl.kernel,
    out_type=jax.ShapeDtypeStruct((BATCH * T_PAD,), jnp.float32),
    mesh=plsc.VectorSubcoreMesh(
        core_axis_name="c", subcore_axis_name="s"),
    scratch_types=[
        pltpu.VMEM((SEQ_W + LANES,), jnp.int32),
        pltpu.VMEM((IDX_W,), jnp.int32),
        pltpu.VMEM((IDX_W,), jnp.float32),
        pltpu.SemaphoreType.DMA,
    ],
)
def _sc_gather(lp_hbm, seq_hbm, out_hbm, seq_v, idx_v, val_v, sem):
    wid = lax.axis_index("s") * NUM_CORES + lax.axis_index("c")
    pltpu.sync_copy(seq_hbm.at[pl.ds(wid * SEQ_W, SEQ_W)],
                    seq_v.at[pl.ds(0, SEQ_W)])

    def row_body(b, carry):
        src_off = b * T_STEPS
        dst_off = b * T_PAD
        for tc in range(T_PAD // LANES):
            s_lab = seq_v[pl.ds(src_off + tc * LANES, LANES)]
            t_lab = seq_v[pl.ds(src_off + tc * LANES + 1, LANES)]
            idx = ((s_lab << 10) | t_lab) & (PAD_LABELS * PAD_LABELS - 1)
            idx_v[pl.ds(dst_off + tc * LANES, LANES)] = idx
        return carry

    lax.fori_loop(0, ROWS_W, row_body, 0, unroll=False)

    def fire(c):
        pltpu.async_copy(
            lp_hbm.at[idx_v.at[pl.ds(c * STREAM_LEN, STREAM_LEN)]],
            val_v.at[pl.ds(c * STREAM_LEN, STREAM_LEN)],
            sem,
        )

    def drain():
        pltpu.make_async_copy(
            lp_hbm.at[pl.ds(0, STREAM_LEN)],
            val_v.at[pl.ds(0, STREAM_LEN)],
            sem,
        ).wait()

    def stream_body(c, carry):
        fire(c)

        @pl.when(c >= LAG)
        def _():
            drain()

        return carry

    lax.fori_loop(0, N_STREAM, stream_body, 0, unroll=False)
    for _ in range(LAG):
        drain()

    pltpu.sync_copy(val_v, out_hbm.at[pl.ds(wid * IDX_W, IDX_W)])


def kernel(label_sequences, source_embeddings, target_embeddings):
    seq = label_sequences.astype(jnp.int32).reshape(-1)
    s_pad = jnp.pad(source_embeddings, ((0, PAD_LABELS - N_LABELS), (0, 0)))
    t_pad = jnp.pad(target_embeddings, ((0, PAD_LABELS - N_LABELS), (0, 0)))
    table = _tc_table(s_pad, t_pad)
    flat = _sc_gather(table.reshape(-1), seq)
    return flat.reshape(BATCH, T_PAD)[:, :T_OUT]


# speedup vs baseline: 1.1766x; 1.1766x over previous
import functools

import jax
import jax.numpy as jnp
from jax import lax
from jax.experimental import pallas as pl
from jax.experimental.pallas import tpu as pltpu
from jax.experimental.pallas import tpu_sc as plsc

N_LABELS = 1000
PAD_LABELS = 1024
EMBED = 64
BATCH = 4096
T_STEPS = 200
T_OUT = T_STEPS - 1
T_PAD = 208
LANES = 16
NUM_CORES = 2
NUM_SUBCORES = 16
NW = NUM_CORES * NUM_SUBCORES
ROWS_W = BATCH // NW
SEQ_W = ROWS_W * T_STEPS
IDX_W = ROWS_W * T_PAD
STREAM_LEN = 128
N_STREAM = IDX_W // STREAM_LEN
LAG = 8
ROW_BLOCK = 128

NEG = -1e30


def _tc_table_body(s_ref, t_ref, o_ref):
    s = s_ref[...]
    t = t_ref[...]
    logits = lax.dot_general(
        s, t, (((1,), (1,)), ((), ())),
        preferred_element_type=jnp.float32,
        precision=lax.Precision.HIGHEST,
    )
    col = lax.broadcasted_iota(jnp.int32, logits.shape, 1)
    logits = jnp.where(col < N_LABELS, logits, NEG)
    m = jnp.max(logits, axis=1, keepdims=True)
    lse = jnp.log(jnp.sum(jnp.exp(logits - m), axis=1, keepdims=True)) + m
    o_ref[...] = logits - lse


def _tc_table(s_pad, t_pad):
    return pl.pallas_call(
        _tc_table_body,
        grid=(PAD_LABELS // ROW_BLOCK,),
        in_specs=[
            pl.BlockSpec((ROW_BLOCK, EMBED), lambda i: (i, 0)),
            pl.BlockSpec((PAD_LABELS, EMBED), lambda i: (0, 0)),
        ],
        out_specs=pl.BlockSpec((ROW_BLOCK, PAD_LABELS), lambda i: (i, 0)),
        out_shape=jax.ShapeDtypeStruct((PAD_LABELS, PAD_LABELS), jnp.float32),
    )(s_pad, t_pad)


@functools.partial(
    pl.kernel,
    out_type=jax.ShapeDtypeStruct((BATCH * T_PAD,), jnp.float32),
    mesh=plsc.VectorSubcoreMesh(
        core_axis_name="c", subcore_axis_name="s"),
    scratch_types=[
        pltpu.VMEM((SEQ_W + LANES,), jnp.int32),
        pltpu.VMEM((IDX_W,), jnp.int32),
        pltpu.VMEM((IDX_W,), jnp.float32),
        pltpu.SemaphoreType.DMA,
    ],
)
def _sc_gather(lp_hbm, seq_hbm, out_hbm, seq_v, idx_v, val_v, sem):
    wid = lax.axis_index("s") * NUM_CORES + lax.axis_index("c")
    pltpu.sync_copy(seq_hbm.at[pl.ds(wid * SEQ_W, SEQ_W)],
                    seq_v.at[pl.ds(0, SEQ_W)])

    def row_body(b, carry):
        src_off = b * T_STEPS
        dst_off = b * T_PAD
        for tc in range(T_PAD // LANES):
            s_lab = seq_v[pl.ds(src_off + tc * LANES, LANES)]
            t_lab = seq_v[pl.ds(src_off + tc * LANES + 1, LANES)]
            idx = ((s_lab << 10) | t_lab) & (PAD_LABELS * PAD_LABELS - 1)
            idx_v[pl.ds(dst_off + tc * LANES, LANES)] = idx
        return carry

    lax.fori_loop(0, ROWS_W, row_body, 0, unroll=False)

    pltpu.async_copy(lp_hbm.at[idx_v], val_v, sem).wait()

    pltpu.sync_copy(val_v, out_hbm.at[pl.ds(wid * IDX_W, IDX_W)])


def kernel(label_sequences, source_embeddings, target_embeddings):
    seq = label_sequences.astype(jnp.int32).reshape(-1)
    s_pad = jnp.pad(source_embeddings, ((0, PAD_LABELS - N_LABELS), (0, 0)))
    t_pad = jnp.pad(target_embeddings, ((0, PAD_LABELS - N_LABELS), (0, 0)))
    table = _tc_table(s_pad, t_pad)
    flat = _sc_gather(table.reshape(-1), seq)
    return flat.reshape(BATCH, T_PAD)[:, :T_OUT]


# speedup vs baseline: 1.3532x; 1.1500x over previous
import functools

import jax
import jax.numpy as jnp
from jax import lax
from jax.experimental import pallas as pl
from jax.experimental.pallas import tpu as pltpu
from jax.experimental.pallas import tpu_sc as plsc

N_LABELS = 1000
PAD_LABELS = 1024
EMBED = 64
BATCH = 4096
T_STEPS = 200
T_OUT = T_STEPS - 1
T_PAD = 208
LANES = 16
NUM_CORES = 2
NUM_SUBCORES = 16
NW = NUM_CORES * NUM_SUBCORES
ROWS_W = BATCH // NW
SEQ_W = ROWS_W * T_STEPS
IDX_W = ROWS_W * T_PAD
N_GROUP = 4
ROWS_G = ROWS_W // N_GROUP
IDX_G = ROWS_G * T_PAD
ROW_BLOCK = 128

NEG = -1e30


def _tc_table_body(s_ref, t_ref, o_ref):
    s = s_ref[...]
    t = t_ref[...]
    logits = lax.dot_general(
        s, t, (((1,), (1,)), ((), ())),
        preferred_element_type=jnp.float32,
        precision=lax.Precision.HIGHEST,
    )
    col = lax.broadcasted_iota(jnp.int32, logits.shape, 1)
    logits = jnp.where(col < N_LABELS, logits, NEG)
    m = jnp.max(logits, axis=1, keepdims=True)
    lse = jnp.log(jnp.sum(jnp.exp(logits - m), axis=1, keepdims=True)) + m
    o_ref[...] = (logits - lse).reshape(ROW_BLOCK * PAD_LABELS)


def _tc_table(s_pad, t_pad):
    return pl.pallas_call(
        _tc_table_body,
        grid=(PAD_LABELS // ROW_BLOCK,),
        in_specs=[
            pl.BlockSpec((ROW_BLOCK, EMBED), lambda i: (i, 0)),
            pl.BlockSpec((PAD_LABELS, EMBED), lambda i: (0, 0)),
        ],
        out_specs=pl.BlockSpec((ROW_BLOCK * PAD_LABELS,), lambda i: (i,)),
        out_shape=jax.ShapeDtypeStruct((PAD_LABELS * PAD_LABELS,), jnp.float32),
    )(s_pad, t_pad)


@functools.partial(
    pl.kernel,
    out_type=jax.ShapeDtypeStruct((BATCH * T_PAD,), jnp.float32),
    mesh=plsc.VectorSubcoreMesh(
        core_axis_name="c", subcore_axis_name="s"),
    scratch_types=[
        pltpu.VMEM((SEQ_W + LANES,), jnp.int32),
        pltpu.VMEM((IDX_W,), jnp.int32),
        pltpu.VMEM((IDX_W,), jnp.float32),
        pltpu.SemaphoreType.DMA,
    ],
)
def _sc_gather(lp_hbm, seq_hbm, out_hbm, seq_v, idx_v, val_v, sem):
    wid = lax.axis_index("s") * NUM_CORES + lax.axis_index("c")
    pltpu.sync_copy(seq_hbm.at[pl.ds(wid * SEQ_W, SEQ_W)],
                    seq_v.at[pl.ds(0, SEQ_W)])

    def row_body(b, carry):
        src_off = b * T_STEPS
        dst_off = b * T_PAD
        for tc in range(T_PAD // LANES):
            s_lab = seq_v[pl.ds(src_off + tc * LANES, LANES)]
            t_lab = seq_v[pl.ds(src_off + tc * LANES + 1, LANES)]
            idx = ((s_lab << 10) | t_lab) & (PAD_LABELS * PAD_LABELS - 1)
            idx_v[pl.ds(dst_off + tc * LANES, LANES)] = idx
        return carry

    for g in range(N_GROUP):
        lax.fori_loop(g * ROWS_G, (g + 1) * ROWS_G, row_body, 0,
                      unroll=False)
        pltpu.async_copy(
            lp_hbm.at[idx_v.at[pl.ds(g * IDX_G, IDX_G)]],
            val_v.at[pl.ds(g * IDX_G, IDX_G)],
            sem,
        )

    pltpu.make_async_copy(
        lp_hbm.at[pl.ds(0, IDX_W)],
        val_v,
        sem,
    ).wait()
    pltpu.sync_copy(val_v, out_hbm.at[pl.ds(wid * IDX_W, IDX_W)])


def kernel(label_sequences, source_embeddings, target_embeddings):
    seq = label_sequences.astype(jnp.int32).reshape(-1)
    s_pad = jnp.pad(source_embeddings, ((0, PAD_LABELS - N_LABELS), (0, 0)))
    t_pad = jnp.pad(target_embeddings, ((0, PAD_LABELS - N_LABELS), (0, 0)))
    table = _tc_table(s_pad, t_pad)
    flat = _sc_gather(table, seq)
    return flat.reshape(BATCH, T_PAD)[:, :T_OUT]
